# baseline (speedup 1.0000x reference)
import jax
import jax.numpy as jnp
from jax import lax
from jax.experimental import pallas as pl
from jax.experimental.pallas import tpu as pltpu
from jax.experimental.pallas import tpu_sc as plsc

N = 6_400_000
NCH = 100_000
NC = 2
NS = 16
NW = NC * NS
B = 8_000
NT = N // B
NTP = 896
CP = 100_352
CPW = CP // NW
PTN = 512
ELN = PTN * 16 + 16


def _sc_body(x_hbm, t_hbm, ids_hbm, sums_o, cnts_o, cht_o,
             idp_a, idp_b, x_a, x_b, t_a, t_b, el_v, pt_v, p_v,
             acc_s, acc_c, acc_t, buf_v, idx_v, sem):
    w = lax.axis_index("c") * NS + lax.axis_index("s")
    iota16 = lax.iota(jnp.int32, 16)
    z16 = jnp.zeros((16,), jnp.float32)

    for j in range(NTP // 128):
        for l in range(8):
            t0 = j * 128 + l * 16
            gi = jnp.minimum((t0 + iota16 + 1) * B - 1, N - 1)
            idx_v[j, pl.ds(l * 16, 16)] = gi
    for j in range(NTP // 128):
        pltpu.async_copy(ids_hbm.at[idx_v.at[j]],
                         buf_v.at[pl.ds(j * 128, 128)], sem).wait()

    lo_ch = w * CPW
    hi_ch = lo_ch + CPW

    def _bsearch(target):
        def it(_, lohi):
            lo, hi = lohi
            mid = (lo + hi) // 2
            go = buf_v[pl.ds(mid, 16)][0] < target
            lo2 = jnp.where(go, mid + 1, lo)
            hi2 = jnp.where(go, hi, mid)
            done = lo >= hi
            return (jnp.where(done, lo, lo2), jnp.where(done, hi, hi2))
        lo, _ = lax.fori_loop(0, 10, it, (jnp.int32(0), jnp.int32(NT)))
        return lo

    t_lo = _bsearch(lo_ch)
    t_hi = _bsearch(hi_ch)
    nb = jnp.maximum(jnp.minimum(t_hi + 1, NT) - t_lo, 0)

    def zr(i, _):
        acc_s[pl.ds(i * 16, 16)] = z16
        acc_c[pl.ds(i * 16, 16)] = z16
        acc_t[pl.ds(i * 16, 16)] = z16
        return 0
    lax.fori_loop(0, CPW // 16, zr, 0)

    def _issue(g, idp, xv, tv_):
        base = (t_lo + g) * B
        pltpu.async_copy(ids_hbm.at[pl.ds(base, B)], idp.at[pl.ds(16, B)],
                         sem)
        pltpu.async_copy(x_hbm.at[pl.ds(base, B)], xv, sem)
        pltpu.async_copy(t_hbm.at[pl.ds(base, B)], tv_, sem)

    def _drain(g, idp, xv, tv_):
        base = (t_lo + g) * B
        pltpu.make_async_copy(ids_hbm.at[pl.ds(base, B)],
                              idp.at[pl.ds(16, B)], sem).wait()
        pltpu.make_async_copy(x_hbm.at[pl.ds(base, B)], xv, sem).wait()
        pltpu.make_async_copy(t_hbm.at[pl.ds(base, B)], tv_, sem).wait()

    def _proc(g, idp, xv, tv_):
        t = t_lo + g
        prev_id = jnp.where(t > 0, buf_v[pl.ds(jnp.maximum(t - 1, 0), 16)][0],
                            -1)
        tb0 = idp[pl.ds(16, 16)][0] != prev_id
        idp[pl.ds(0, 16)] = jnp.full((16,), -1, jnp.int32)

        def pa(o, off):
            x = xv[pl.ds(o, 16)]
            el_v[pl.ds(o, 16)] = plsc.cumsum(x)
            cur = idp[pl.ds(o + 16, 16)]
            prv = idp[pl.ds(o + 15, 16)]
            m = cur != prv
            plsc.store_compressed(p_v.at[pl.ds(off, 16)], o + iota16, mask=m)
            return off + plsc.all_reduce_population_count(m)[0]
        K = plsc.parallel_loop(0, B, step=16, unroll=8,
                               carry=jnp.int32(0))(pa)
        p_v[pl.ds(K, 16)] = jnp.full((16,), B, jnp.int32)

        def pa2(jo, cvs):
            ks = jo + iota16
            tk = plsc.load_gather(el_v, [ks * 16 + 15])
            s2 = plsc.cumsum(tk)
            pt_v[pl.ds(jo, 16)] = s2 - tk + cvs
            return cvs + s2[15]
        plsc.parallel_loop(0, PTN, step=16, carry=jnp.float32(0.0))(pa2)

        def eat(pos):
            base = plsc.load_gather(pt_v, [lax.shift_right_logical(pos, 4)])
            frac = plsc.load_gather(el_v, [jnp.maximum(pos - 1, 0)])
            return base + jnp.where((pos & 15) == 0, 0.0, frac)

        def pb(ko):
            p0 = p_v[pl.ds(ko, 16)]
            p1 = p_v[pl.ds(ko + 1, 16)]
            seg = eat(p1) - eat(p0)
            cnt = (p1 - p0).astype(jnp.float32)
            cid = plsc.load_gather(idp, [jnp.minimum(p0 + 16, B + 15)])
            tv = plsc.load_gather(tv_, [jnp.minimum(p0, B - 1)])
            kvec = ko + iota16
            valid = (cid >= lo_ch) & (cid < hi_ch) & (kvec < K)
            li = jnp.clip(cid - lo_ch, 0, CPW - 1)
            plsc.addupdate_scatter(acc_s, [li], seg, mask=valid)
            plsc.addupdate_scatter(acc_c, [li], cnt, mask=valid)
            tmask = valid & ((kvec != 0) | tb0)
            plsc.addupdate_scatter(acc_t, [li], tv, mask=tmask)
        plsc.parallel_loop(0, ((K + 15) // 16) * 16, step=16)(pb)

    for o in range(B, ELN, 16):
        el_v[pl.ds(o, 16)] = z16

    pl.when(nb > 0)(lambda: _issue(0, idp_a, x_a, t_a))

    def pairblk(q, _):
        g0 = 2 * q

        def do0():
            _drain(g0, idp_a, x_a, t_a)
            pl.when(g0 + 1 < nb)(lambda: _issue(g0 + 1, idp_b, x_b, t_b))
            _proc(g0, idp_a, x_a, t_a)
        do0()

        def do1():
            g1 = g0 + 1
            _drain(g1, idp_b, x_b, t_b)
            pl.when(g1 + 1 < nb)(lambda: _issue(g1 + 1, idp_a, x_a, t_a))
            _proc(g1, idp_b, x_b, t_b)
        pl.when(g0 + 1 < nb)(do1)
        return 0
    lax.fori_loop(0, (nb + 1) // 2, pairblk, 0)

    cbase = w * CPW
    pltpu.sync_copy(acc_s, sums_o.at[pl.ds(cbase, CPW)])
    pltpu.sync_copy(acc_c, cnts_o.at[pl.ds(cbase, CPW)])
    pltpu.sync_copy(acc_t, cht_o.at[pl.ds(cbase, CPW)])


_sc_call = pl.kernel(
    _sc_body,
    out_type=(jax.ShapeDtypeStruct((CP,), jnp.float32),) * 3,
    mesh=plsc.VectorSubcoreMesh(core_axis_name="c", subcore_axis_name="s"),
    compiler_params=pltpu.CompilerParams(needs_layout_passes=False),
    scratch_types=[
        pltpu.VMEM((B + 16,), jnp.int32),
        pltpu.VMEM((B + 16,), jnp.int32),
        pltpu.VMEM((B,), jnp.float32),
        pltpu.VMEM((B,), jnp.float32),
        pltpu.VMEM((B,), jnp.float32),
        pltpu.VMEM((B,), jnp.float32),
        pltpu.VMEM((ELN,), jnp.float32),
        pltpu.VMEM((PTN,), jnp.float32),
        pltpu.VMEM((B + 32,), jnp.int32),
        pltpu.VMEM((CPW,), jnp.float32),
        pltpu.VMEM((CPW,), jnp.float32),
        pltpu.VMEM((CPW,), jnp.float32),
        pltpu.VMEM((NTP,), jnp.int32),
        pltpu.VMEM((NTP // 128, 128), jnp.int32),
        pltpu.SemaphoreType.DMA,
    ],
)


def _sig_body(x_ref, o_ref):
    x = x_ref[...]
    o_ref[...] = 1.0 / (1.0 + jnp.exp(-x))


def _fin_body(s_ref, c_ref, t_ref, o_ref):
    s = s_ref[...]
    c = c_ref[...]
    t = t_ref[...]
    present = c > 0.0
    mean = s / jnp.maximum(c, 1.0)
    lp = jnp.maximum(jnp.log(mean), -100.0)
    l1 = jnp.maximum(jnp.log(1.0 - mean), -100.0)
    per = -(t * lp + (1.0 - t) * l1)
    per = jnp.where(present, per, 0.0)
    num = jnp.sum(per)
    den = jnp.maximum(jnp.sum(present.astype(jnp.float32)), 1.0)
    o_ref[...] = (num / den).reshape(1, 1)


def kernel(output, target, ch_ids):
    ids32 = ch_ids.astype(jnp.int32)
    probs = pl.pallas_call(
        _sig_body,
        out_shape=jax.ShapeDtypeStruct((N // 128, 128), jnp.float32),
        grid=(10,),
        in_specs=[pl.BlockSpec((N // 128 // 10, 128), lambda i: (i, 0))],
        out_specs=pl.BlockSpec((N // 128 // 10, 128), lambda i: (i, 0)),
    )(output.reshape(N // 128, 128)).reshape(N)
    sums, cnts, cht = _sc_call(probs, target, ids32)
    out = pl.pallas_call(
        _fin_body,
        out_shape=jax.ShapeDtypeStruct((1, 1), jnp.float32),
    )(sums.reshape(CP // 128, 128),
      cnts.reshape(CP // 128, 128),
      cht.reshape(CP // 128, 128))
    return out[0, 0]


# speedup vs baseline: 1.0031x; 1.0031x over previous
import jax
import jax.numpy as jnp
from jax import lax
from jax.experimental import pallas as pl
from jax.experimental.pallas import tpu as pltpu
from jax.experimental.pallas import tpu_sc as plsc

N = 6_400_000
NCH = 100_000
NC = 2
NS = 16
NW = NC * NS
B = 8_000
NT = N // B
NTP = 896
CP = 100_352
CPW = CP // NW
PTN = 512
ELN = PTN * 16 + 16


def _sc_body(x_hbm, t_hbm, ids_hbm, sums_o, cnts_o, cht_o,
             idp_a, idp_b, x_a, x_b, t_a, t_b, el_v, pt_v, p_v,
             acc_s, acc_c, acc_t, buf_v, idx_v, sem):
    w = lax.axis_index("c") * NS + lax.axis_index("s")
    iota16 = lax.iota(jnp.int32, 16)
    z16 = jnp.zeros((16,), jnp.float32)

    for j in range(NTP // 128):
        for l in range(8):
            t0 = j * 128 + l * 16
            gi = jnp.minimum((t0 + iota16 + 1) * B - 1, N - 1)
            idx_v[j, pl.ds(l * 16, 16)] = gi
    for j in range(NTP // 128):
        pltpu.async_copy(ids_hbm.at[idx_v.at[j]],
                         buf_v.at[pl.ds(j * 128, 128)], sem).wait()

    lo_ch = w * CPW
    hi_ch = lo_ch + CPW

    def _bsearch(target):
        def it(_, lohi):
            lo, hi = lohi
            mid = (lo + hi) // 2
            go = buf_v[pl.ds(mid, 16)][0] < target
            lo2 = jnp.where(go, mid + 1, lo)
            hi2 = jnp.where(go, hi, mid)
            done = lo >= hi
            return (jnp.where(done, lo, lo2), jnp.where(done, hi, hi2))
        lo, _ = lax.fori_loop(0, 10, it, (jnp.int32(0), jnp.int32(NT)))
        return lo

    t_lo = _bsearch(lo_ch)
    t_hi = _bsearch(hi_ch)
    nb = jnp.maximum(jnp.minimum(t_hi + 1, NT) - t_lo, 0)

    def zr(i, _):
        acc_s[pl.ds(i * 16, 16)] = z16
        acc_c[pl.ds(i * 16, 16)] = z16
        acc_t[pl.ds(i * 16, 16)] = z16
        return 0
    lax.fori_loop(0, CPW // 16, zr, 0)

    def _issue(g, idp, xv, tv_):
        base = (t_lo + g) * B
        pltpu.async_copy(ids_hbm.at[pl.ds(base, B)], idp.at[pl.ds(16, B)],
                         sem)
        pltpu.async_copy(x_hbm.at[pl.ds(base, B)], xv, sem)
        pltpu.async_copy(t_hbm.at[pl.ds(base, B)], tv_, sem)

    def _drain(g, idp, xv, tv_):
        base = (t_lo + g) * B
        pltpu.make_async_copy(ids_hbm.at[pl.ds(base, B)],
                              idp.at[pl.ds(16, B)], sem).wait()
        pltpu.make_async_copy(x_hbm.at[pl.ds(base, B)], xv, sem).wait()
        pltpu.make_async_copy(t_hbm.at[pl.ds(base, B)], tv_, sem).wait()

    def _proc(g, idp, xv, tv_):
        t = t_lo + g
        prev_id = jnp.where(t > 0, buf_v[pl.ds(jnp.maximum(t - 1, 0), 16)][0],
                            -1)
        tb0 = idp[pl.ds(16, 16)][0] != prev_id
        idp[pl.ds(0, 16)] = jnp.full((16,), -1, jnp.int32)

        def pa(o, off):
            x = xv[pl.ds(o, 16)]
            el_v[pl.ds(o, 16)] = plsc.cumsum(x)
            cur = idp[pl.ds(o + 16, 16)]
            prv = idp[pl.ds(o + 15, 16)]
            m = cur != prv
            plsc.store_compressed(p_v.at[pl.ds(off, 16)], o + iota16, mask=m)
            return off + plsc.all_reduce_population_count(m)[0]
        K = plsc.parallel_loop(0, B, step=16, unroll=8,
                               carry=jnp.int32(0))(pa)
        p_v[pl.ds(K, 16)] = jnp.full((16,), B, jnp.int32)

        def pa2(jo, cvs):
            ks = jo + iota16
            tk = plsc.load_gather(el_v, [ks * 16 + 15])
            s2 = plsc.cumsum(tk)
            pt_v[pl.ds(jo, 16)] = s2 - tk + cvs
            return cvs + s2[15]
        plsc.parallel_loop(0, PTN, step=16, carry=jnp.float32(0.0))(pa2)

        def eat(pos):
            base = plsc.load_gather(pt_v, [lax.shift_right_logical(pos, 4)])
            frac = plsc.load_gather(el_v, [jnp.maximum(pos - 1, 0)])
            return base + jnp.where((pos & 15) == 0, 0.0, frac)

        def pb(ko):
            p0 = p_v[pl.ds(ko, 16)]
            p1 = p_v[pl.ds(ko + 1, 16)]
            seg = eat(p1) - eat(p0)
            cnt = (p1 - p0).astype(jnp.float32)
            cid = plsc.load_gather(idp, [jnp.minimum(p0 + 16, B + 15)])
            tv = plsc.load_gather(tv_, [jnp.minimum(p0, B - 1)])
            kvec = ko + iota16
            valid = (cid >= lo_ch) & (cid < hi_ch) & (kvec < K)
            li = jnp.clip(cid - lo_ch, 0, CPW - 1)
            plsc.addupdate_scatter(acc_s, [li], seg, mask=valid)
            plsc.addupdate_scatter(acc_c, [li], cnt, mask=valid)
            tmask = valid & ((kvec != 0) | tb0)
            plsc.addupdate_scatter(acc_t, [li], tv, mask=tmask)
        plsc.parallel_loop(0, ((K + 15) // 16) * 16, step=16)(pb)

    for o in range(B, ELN, 16):
        el_v[pl.ds(o, 16)] = z16

    pl.when(nb > 0)(lambda: _issue(0, idp_a, x_a, t_a))

    def pairblk(q, _):
        g0 = 2 * q

        def do0():
            _drain(g0, idp_a, x_a, t_a)
            pl.when(g0 + 1 < nb)(lambda: _issue(g0 + 1, idp_b, x_b, t_b))
            _proc(g0, idp_a, x_a, t_a)
        do0()

        def do1():
            g1 = g0 + 1
            _drain(g1, idp_b, x_b, t_b)
            pl.when(g1 + 1 < nb)(lambda: _issue(g1 + 1, idp_a, x_a, t_a))
            _proc(g1, idp_b, x_b, t_b)
        pl.when(g0 + 1 < nb)(do1)
        return 0
    lax.fori_loop(0, (nb + 1) // 2, pairblk, 0)

    cbase = w * CPW
    pltpu.sync_copy(acc_s, sums_o.at[pl.ds(cbase, CPW)])
    pltpu.sync_copy(acc_c, cnts_o.at[pl.ds(cbase, CPW)])
    pltpu.sync_copy(acc_t, cht_o.at[pl.ds(cbase, CPW)])


_sc_call = pl.kernel(
    _sc_body,
    out_type=(jax.ShapeDtypeStruct((CP,), jnp.float32),) * 3,
    mesh=plsc.VectorSubcoreMesh(core_axis_name="c", subcore_axis_name="s",
                                num_cores=NC, num_subcores=NS),
    compiler_params=pltpu.CompilerParams(needs_layout_passes=False),
    scratch_types=[
        pltpu.VMEM((B + 16,), jnp.int32),
        pltpu.VMEM((B + 16,), jnp.int32),
        pltpu.VMEM((B,), jnp.float32),
        pltpu.VMEM((B,), jnp.float32),
        pltpu.VMEM((B,), jnp.float32),
        pltpu.VMEM((B,), jnp.float32),
        pltpu.VMEM((ELN,), jnp.float32),
        pltpu.VMEM((PTN,), jnp.float32),
        pltpu.VMEM((B + 32,), jnp.int32),
        pltpu.VMEM((CPW,), jnp.float32),
        pltpu.VMEM((CPW,), jnp.float32),
        pltpu.VMEM((CPW,), jnp.float32),
        pltpu.VMEM((NTP,), jnp.int32),
        pltpu.VMEM((NTP // 128, 128), jnp.int32),
        pltpu.SemaphoreType.DMA,
    ],
)


def _sig_body(x_ref, o_ref):
    x = x_ref[...]
    o_ref[...] = 1.0 / (1.0 + jnp.exp(-x))


def _fin_body(s_ref, c_ref, t_ref, o_ref):
    s = s_ref[...]
    c = c_ref[...]
    t = t_ref[...]
    present = c > 0.0
    mean = s / jnp.maximum(c, 1.0)
    lp = jnp.maximum(jnp.log(mean), -100.0)
    l1 = jnp.maximum(jnp.log(1.0 - mean), -100.0)
    per = -(t * lp + (1.0 - t) * l1)
    per = jnp.where(present, per, 0.0)
    num = jnp.sum(per)
    den = jnp.maximum(jnp.sum(present.astype(jnp.float32)), 1.0)
    o_ref[...] = (num / den).reshape(1, 1)


def kernel(output, target, ch_ids):
    ids32 = ch_ids.astype(jnp.int32)
    probs = pl.pallas_call(
        _sig_body,
        out_shape=jax.ShapeDtypeStruct((N // 128, 128), jnp.float32),
        grid=(10,),
        in_specs=[pl.BlockSpec((N // 128 // 10, 128), lambda i: (i, 0))],
        out_specs=pl.BlockSpec((N // 128 // 10, 128), lambda i: (i, 0)),
    )(output.reshape(N // 128, 128)).reshape(N)
    sums, cnts, cht = _sc_call(probs, target, ids32)
    out = pl.pallas_call(
        _fin_body,
        out_shape=jax.ShapeDtypeStruct((1, 1), jnp.float32),
    )(sums.reshape(CP // 128, 128),
      cnts.reshape(CP // 128, 128),
      cht.reshape(CP // 128, 128))
    return out[0, 0]
